# Initial kernel scaffold; baseline (speedup 1.0000x reference)
import functools

import jax
import jax.numpy as jnp
from jax.experimental import pallas as pl
from jax.experimental.pallas import tpu as pltpu
from jax.experimental.pallas import tpu_sc as plsc

_B, _N, _D, _K = 32, 1024, 256, 1024
_BN = _B * _N
_BM = 1024
_NT = _BN // _BM
_GW = 64


def _dist_kernel(x_ref, ct_ref, dist_ref, ind_ref, acc_ref):
    i = pl.program_id(0)
    x = x_ref[...]
    ct = ct_ref[...]
    dots = jnp.dot(x, ct, preferred_element_type=jnp.float32)
    z_sq = jnp.sum(x * x, axis=1, keepdims=True)
    c_sq = jnp.sum(ct * ct, axis=0, keepdims=True)
    dist = -(z_sq - 2.0 * dots + c_sq)
    dist_ref[...] = dist
    m = jnp.max(dist, axis=1)
    iota = jax.lax.broadcasted_iota(jnp.int32, dist.shape, 1)
    cand = jnp.where(dist == m[:, None], iota, _K)
    ind_ref[0, 0, :] = jnp.min(cand, axis=1)

    @pl.when(i == 0)
    def _():
        acc_ref[0, 0] = 0.0

    acc_ref[0, 0] += jnp.sum(m)


def _tc_dist(flat, cbt):
    return pl.pallas_call(
        _dist_kernel,
        grid=(_NT,),
        in_specs=[
            pl.BlockSpec((_BM, _D), lambda i: (i, 0)),
            pl.BlockSpec((_D, _K), lambda i: (0, 0)),
        ],
        out_specs=[
            pl.BlockSpec((_BM, _K), lambda i: (i, 0)),
            pl.BlockSpec((1, 1, _BM), lambda i: (i, 0, 0)),
            pl.BlockSpec((1, 1), lambda i: (0, 0)),
        ],
        out_shape=[
            jax.ShapeDtypeStruct((_BN, _K), jnp.float32),
            jax.ShapeDtypeStruct((_NT, 1, _BM), jnp.int32),
            jax.ShapeDtypeStruct((1, 1), jnp.float32),
        ],
    )(flat, cbt)


def _sc_gather(codebook, indices):
    mesh = plsc.VectorSubcoreMesh(core_axis_name="core",
                                  subcore_axis_name="subcore")

    @functools.partial(
        pl.kernel,
        out_type=jax.ShapeDtypeStruct((_BN, _D), codebook.dtype),
        mesh=mesh,
    )
    def gather_kernel(cb_hbm, i_hbm, o_hbm):
        def body(i_vmem, o_vmem):
            pltpu.sync_copy(cb_hbm.at[i_vmem.at[0]], o_vmem)

        pltpu.emit_pipeline(
            body,
            grid=(_BN // _GW,),
            in_specs=[pl.BlockSpec((1, _GW), index_map=lambda i: (0, i))],
            out_specs=[pl.BlockSpec((_GW, _D), index_map=lambda i: (i, 0))],
            core_axis_name=("core", "subcore"),
            dimension_semantics=(pltpu.PARALLEL,),
        )(i_hbm, o_hbm)

    return gather_kernel(codebook, indices)


def kernel(z, stochastic, codebook):
    flat = z.reshape(_BN, _D)
    cbt = codebook.T
    dist, ind_tiles, maxsum = _tc_dist(flat, cbt)
    embed_ind = ind_tiles.reshape(_B, _N)
    vqloss = -2.0 * maxsum[0, 0] / (_BN * _D)
    zq = _sc_gather(codebook, ind_tiles.reshape(1, _BN))
    return (zq.reshape(_B, _N, _D), embed_ind, vqloss,
            dist.reshape(_B, _N, _K))


# speedup vs baseline: 5.3753x; 5.3753x over previous
import functools

import jax
import jax.numpy as jnp
from jax.experimental import pallas as pl
from jax.experimental.pallas import tpu as pltpu
from jax.experimental.pallas import tpu_sc as plsc

_B, _N, _D, _K = 32, 1024, 256, 1024
_BN = _B * _N
_BM = 1024
_NT = _BN // _BM
_GW = 128


def _dist_kernel(x_ref, ct_ref, dist_ref, ind_ref, acc_ref):
    i = pl.program_id(0)
    x = x_ref[...]
    ct = ct_ref[...]
    dots = jnp.dot(x, ct, preferred_element_type=jnp.float32)
    z_sq = jnp.sum(x * x, axis=1, keepdims=True)
    c_sq = jnp.sum(ct * ct, axis=0, keepdims=True)
    dist = -(z_sq - 2.0 * dots + c_sq)
    dist_ref[...] = dist
    m = jnp.max(dist, axis=1)
    iota = jax.lax.broadcasted_iota(jnp.int32, dist.shape, 1)
    cand = jnp.where(dist == m[:, None], iota, _K)
    ind_ref[0, 0, :] = jnp.min(cand, axis=1)

    @pl.when(i == 0)
    def _():
        acc_ref[...] = jnp.zeros_like(acc_ref)

    acc_ref[...] += jnp.sum(m)[None, None]


def _tc_dist(flat, cbt):
    return pl.pallas_call(
        _dist_kernel,
        grid=(_NT,),
        in_specs=[
            pl.BlockSpec((_BM, _D), lambda i: (i, 0)),
            pl.BlockSpec((_D, _K), lambda i: (0, 0)),
        ],
        out_specs=[
            pl.BlockSpec((_BM, _K), lambda i: (i, 0)),
            pl.BlockSpec((1, 1, _BM), lambda i: (i, 0, 0)),
            pl.BlockSpec((1, 1), lambda i: (0, 0)),
        ],
        out_shape=[
            jax.ShapeDtypeStruct((_BN, _K), jnp.float32),
            jax.ShapeDtypeStruct((_NT, 1, _BM), jnp.int32),
            jax.ShapeDtypeStruct((1, 1), jnp.float32),
        ],
    )(flat, cbt)


def _sc_gather(codebook, indices):
    mesh = plsc.VectorSubcoreMesh(core_axis_name="core",
                                  subcore_axis_name="subcore")

    @functools.partial(
        pl.kernel,
        out_type=jax.ShapeDtypeStruct((_BN, _D), codebook.dtype),
        mesh=mesh,
    )
    def gather_kernel(cb_hbm, i_hbm, o_hbm):
        def body(i_vmem, o_vmem):
            pltpu.sync_copy(cb_hbm.at[i_vmem.at[0]], o_vmem)

        pltpu.emit_pipeline(
            body,
            grid=(_BN // _GW,),
            in_specs=[pl.BlockSpec((1, _GW), index_map=lambda i: (0, i))],
            out_specs=[pl.BlockSpec((_GW, _D), index_map=lambda i: (i, 0))],
            core_axis_name=("core", "subcore"),
            dimension_semantics=(pltpu.PARALLEL,),
        )(i_hbm, o_hbm)

    return gather_kernel(codebook, indices)


def kernel(z, stochastic, codebook):
    flat = z.reshape(_BN, _D)
    cbt = codebook.T
    dist, ind_tiles, maxsum = _tc_dist(flat, cbt)
    embed_ind = ind_tiles.reshape(_B, _N)
    vqloss = -2.0 * maxsum[0, 0] / (_BN * _D)
    zq = _sc_gather(codebook, ind_tiles.reshape(1, _BN))
    return (zq.reshape(_B, _N, _D), embed_ind, vqloss,
            dist.reshape(_B, _N, _K))


# speedup vs baseline: 5.6676x; 1.0544x over previous
import functools

import jax
import jax.numpy as jnp
from jax.experimental import pallas as pl
from jax.experimental.pallas import tpu as pltpu
from jax.experimental.pallas import tpu_sc as plsc

_B, _N, _D, _K = 32, 1024, 256, 1024
_BN = _B * _N
_BM = 1024
_NT = _BN // _BM
_GW = 128


def _dist_kernel(x_ref, ct_ref, dist_ref, ind_ref, acc_ref):
    x = x_ref[...]
    ct = ct_ref[...]
    dots = jnp.dot(x, ct, preferred_element_type=jnp.float32)
    z_sq = jnp.sum(x * x, axis=1, keepdims=True)
    c_sq = jnp.sum(ct * ct, axis=0, keepdims=True)
    dist = -(z_sq - 2.0 * dots + c_sq)
    dist_ref[...] = dist
    m = jnp.max(dist, axis=1)
    iota = jax.lax.broadcasted_iota(jnp.int32, dist.shape, 1)
    cand = jnp.where(dist == m[:, None], iota, _K)
    ind_ref[0, 0, :] = jnp.min(cand, axis=1)
    acc_ref[...] = jnp.sum(m)[None, None, None]


def _tc_dist(flat, cbt):
    return pl.pallas_call(
        _dist_kernel,
        grid=(_NT,),
        in_specs=[
            pl.BlockSpec((_BM, _D), lambda i: (i, 0)),
            pl.BlockSpec((_D, _K), lambda i: (0, 0)),
        ],
        out_specs=[
            pl.BlockSpec((_BM, _K), lambda i: (i, 0)),
            pl.BlockSpec((1, 1, _BM), lambda i: (i, 0, 0)),
            pl.BlockSpec((1, 1, 1), lambda i: (i, 0, 0)),
        ],
        out_shape=[
            jax.ShapeDtypeStruct((_BN, _K), jnp.float32),
            jax.ShapeDtypeStruct((_NT, 1, _BM), jnp.int32),
            jax.ShapeDtypeStruct((_NT, 1, 1), jnp.float32),
        ],
        compiler_params=pltpu.CompilerParams(
            dimension_semantics=("parallel",)),
    )(flat, cbt)


def _sc_gather(codebook, indices):
    mesh = plsc.VectorSubcoreMesh(core_axis_name="core",
                                  subcore_axis_name="subcore")

    @functools.partial(
        pl.kernel,
        out_type=jax.ShapeDtypeStruct((_BN, _D), codebook.dtype),
        mesh=mesh,
    )
    def gather_kernel(cb_hbm, i_hbm, o_hbm):
        def body(i_vmem, o_vmem):
            pltpu.sync_copy(cb_hbm.at[i_vmem.at[0]], o_vmem)

        pltpu.emit_pipeline(
            body,
            grid=(_BN // _GW,),
            in_specs=[pl.BlockSpec((1, _GW), index_map=lambda i: (0, i))],
            out_specs=[pl.BlockSpec((_GW, _D), index_map=lambda i: (i, 0))],
            core_axis_name=("core", "subcore"),
            dimension_semantics=(pltpu.PARALLEL,),
        )(i_hbm, o_hbm)

    return gather_kernel(codebook, indices)


def kernel(z, stochastic, codebook):
    flat = z.reshape(_BN, _D)
    cbt = codebook.T
    dist, ind_tiles, maxsums = _tc_dist(flat, cbt)
    embed_ind = ind_tiles.reshape(_B, _N)
    vqloss = -2.0 * jnp.sum(maxsums) / (_BN * _D)
    zq = _sc_gather(codebook, ind_tiles.reshape(1, _BN))
    return (zq.reshape(_B, _N, _D), embed_ind, vqloss,
            dist.reshape(_B, _N, _K))


# speedup vs baseline: 6.7140x; 1.1846x over previous
import functools

import jax
import jax.numpy as jnp
from jax.experimental import pallas as pl
from jax.experimental.pallas import tpu as pltpu
from jax.experimental.pallas import tpu_sc as plsc

_B, _N, _D, _K = 32, 1024, 256, 1024
_BN = _B * _N
_BM = 4096
_NT = _BN // _BM
_GW = 128


def _dist_kernel(x_ref, ct2_ref, dist_ref, ind_ref, acc_ref):
    x = x_ref[...]
    ct2 = ct2_ref[...]
    dots2 = jnp.dot(x, ct2, preferred_element_type=jnp.float32)
    z_sq = jnp.sum(x * x, axis=1, keepdims=True)
    c_sq = 0.25 * jnp.sum(ct2 * ct2, axis=0, keepdims=True)
    dist = (dots2 - z_sq) - c_sq
    dist_ref[...] = dist
    m = jnp.max(dist, axis=1)
    iota = jax.lax.broadcasted_iota(jnp.int32, dist.shape, 1)
    cand = jnp.where(dist == m[:, None], iota, _K)
    ind_ref[0, 0, :] = jnp.min(cand, axis=1)
    acc_ref[...] = jnp.sum(m)[None, None, None]


def _tc_dist(flat, cbt):
    return pl.pallas_call(
        _dist_kernel,
        grid=(_NT,),
        in_specs=[
            pl.BlockSpec((_BM, _D), lambda i: (i, 0)),
            pl.BlockSpec((_D, _K), lambda i: (0, 0)),
        ],
        out_specs=[
            pl.BlockSpec((_BM, _K), lambda i: (i, 0)),
            pl.BlockSpec((1, 1, _BM), lambda i: (i, 0, 0)),
            pl.BlockSpec((1, 1, 1), lambda i: (i, 0, 0)),
        ],
        out_shape=[
            jax.ShapeDtypeStruct((_BN, _K), jnp.float32),
            jax.ShapeDtypeStruct((_NT, 1, _BM), jnp.int32),
            jax.ShapeDtypeStruct((_NT, 1, 1), jnp.float32),
        ],
    )(flat, cbt)


_NW = 32
_NCHUNK = _BN // (_NW * _GW)
_NBUF = 3


def _sc_gather(codebook, indices):
    mesh = plsc.VectorSubcoreMesh(core_axis_name="c", subcore_axis_name="s")

    @functools.partial(
        pl.kernel,
        out_type=jax.ShapeDtypeStruct((_BN, _D), codebook.dtype),
        mesh=mesh,
        scratch_types=(
            [pltpu.VMEM((_NCHUNK, _GW), jnp.int32)]
            + [pltpu.VMEM((_GW, _D), codebook.dtype)] * _NBUF
            + [pltpu.SemaphoreType.DMA] * (2 * _NBUF + 1)
        ),
    )
    def gather_kernel(cb_hbm, i_hbm, o_hbm, idx_v,
                      b0, b1, b2, g0, g1, g2, w0, w1, w2, isem):
        bufs, gsems, wsems = [b0, b1, b2], [g0, g1, g2], [w0, w1, w2]
        wid = jax.lax.axis_index("s") * 2 + jax.lax.axis_index("c")
        base = wid * (_NCHUNK * _GW)
        pltpu.async_copy(i_hbm.at[wid], idx_v, isem).wait()
        gh = [None] * _NCHUNK
        for c in range(min(_NBUF, _NCHUNK)):
            gh[c] = pltpu.make_async_copy(
                cb_hbm.at[idx_v.at[c]], bufs[c], gsems[c])
            gh[c].start()
        for c in range(_NCHUNK):
            j = c % _NBUF
            gh[c].wait()
            wh = pltpu.make_async_copy(
                bufs[j], o_hbm.at[pl.ds(base + c * _GW, _GW)], wsems[j])
            wh.start()
            nc = c + _NBUF
            if nc < _NCHUNK:
                wh.wait()
                gh[nc] = pltpu.make_async_copy(
                    cb_hbm.at[idx_v.at[nc]], bufs[j], gsems[j])
                gh[nc].start()
            else:
                wh.wait()

    return gather_kernel(codebook, indices)


def kernel(z, stochastic, codebook):
    flat = z.reshape(_BN, _D)
    cbt = (2.0 * codebook).T
    dist, ind_tiles, maxsums = _tc_dist(flat, cbt)
    embed_ind = ind_tiles.reshape(_B, _N)
    vqloss = -2.0 * jnp.sum(maxsums) / (_BN * _D)
    zq = _sc_gather(codebook, ind_tiles.reshape(_NW, _NCHUNK, _GW))
    return (zq.reshape(_B, _N, _D), embed_ind, vqloss,
            dist.reshape(_B, _N, _K))
